# Initial kernel scaffold; baseline (speedup 1.0000x reference)
import jax
import jax.numpy as jnp
from jax.experimental import pallas as pl


def kernel(feat_data, edge_index, inputs1, inputs2, neg, W1, b1, W2, b2):
    raise NotImplementedError("write your pallas kernel here")


# speedup vs baseline: 2.2346x; 2.2346x over previous
import functools

import jax
import jax.numpy as jnp
from jax import lax
from jax.experimental import pallas as pl
from jax.experimental.pallas import tpu as pltpu
from jax.experimental.pallas import tpu_sc as plsc

_N = 10000
_E = 320000
_D = 128
_B = 1024
_NEG = 10

_NC = 2
_NS = 16
_NW = _NC * _NS
_CHUNK = 128
_N_PAD = 10240
_E_PAD = 327680
_CPT = _E_PAD // (_NW * _CHUNK)
_STRIPE = _N_PAD // _NS
_G = _NW * _CHUNK

_mesh = plsc.VectorSubcoreMesh(core_axis_name="c", subcore_axis_name="s")


@functools.partial(
    pl.kernel,
    out_type=jax.ShapeDtypeStruct((_NC, _N_PAD, _D), jnp.float32),
    mesh=_mesh,
    scratch_types=[
        pltpu.VMEM((_CPT, _CHUNK), jnp.int32),
        pltpu.VMEM((_CPT, _CHUNK), jnp.int32),
        pltpu.VMEM((_CHUNK, _D), jnp.float32),
        pltpu.VMEM_SHARED((_N_PAD, _D), jnp.float32),
        pltpu.SemaphoreType.DMA,
    ],
)
def _segsum(table_hbm, src_hbm, dst_hbm, zeros_hbm, out_hbm,
            src_v, dst_v, rows_v, acc, sem):
    c = lax.axis_index("c")
    s = lax.axis_index("s")
    w = c * _NS + s
    pltpu.sync_copy(zeros_hbm.at[pl.ds(s * _STRIPE, _STRIPE)],
                    acc.at[pl.ds(s * _STRIPE, _STRIPE)])
    pltpu.sync_copy(src_hbm.at[w], src_v)
    pltpu.sync_copy(dst_hbm.at[w], dst_v)
    plsc.subcore_barrier()

    def body(j, carry):
        pltpu.async_copy(table_hbm.at[src_v.at[j]], rows_v, sem).wait()
        pltpu.sync_copy(rows_v, acc.at[dst_v.at[j]], add=True)
        return carry

    lax.fori_loop(0, _CPT, body, 0)
    plsc.subcore_barrier()
    pltpu.sync_copy(acc.at[pl.ds(s * _STRIPE, _STRIPE)],
                    out_hbm.at[c, pl.ds(s * _STRIPE, _STRIPE)])


@functools.partial(
    pl.kernel,
    out_type=(
        jax.ShapeDtypeStruct((_G, _D), jnp.float32),
        jax.ShapeDtypeStruct((_G, _D), jnp.float32),
        jax.ShapeDtypeStruct((_G, _D), jnp.float32),
    ),
    mesh=_mesh,
    scratch_types=[
        pltpu.VMEM((_CHUNK,), jnp.int32),
        pltpu.VMEM((_CHUNK, _D), jnp.float32),
        pltpu.SemaphoreType.DMA,
    ],
)
def _gather3(feat_hbm, p0_hbm, p1_hbm, idx_hbm, gf_hbm, g0_hbm, g1_hbm,
             idx_v, rows_v, sem):
    c = lax.axis_index("c")
    s = lax.axis_index("s")
    w = c * _NS + s
    base = w * _CHUNK
    pltpu.sync_copy(idx_hbm.at[pl.ds(base, _CHUNK)], idx_v)
    pltpu.async_copy(feat_hbm.at[idx_v], rows_v, sem).wait()
    pltpu.sync_copy(rows_v, gf_hbm.at[pl.ds(base, _CHUNK)])
    pltpu.async_copy(p0_hbm.at[idx_v], rows_v, sem).wait()
    pltpu.sync_copy(rows_v, g0_hbm.at[pl.ds(base, _CHUNK)])
    pltpu.async_copy(p1_hbm.at[idx_v], rows_v, sem).wait()
    pltpu.sync_copy(rows_v, g1_hbm.at[pl.ds(base, _CHUNK)])


def _layer1_body(feat_ref, p0_ref, p1_ref, w1a_ref, w1b_ref, b1_ref, out_ref):
    ns = p0_ref[...] + p1_ref[...]
    acc = jnp.dot(feat_ref[...], w1a_ref[...],
                  preferred_element_type=jnp.float32)
    acc = acc + jnp.dot(ns, w1b_ref[...], preferred_element_type=jnp.float32)
    out_ref[...] = jnp.tanh(acc + b1_ref[...])


_LBLK = 1024


def _layer1(feat_pad, p0, p1, w1a, w1b, b1r):
    grid = (_N_PAD // _LBLK,)
    row_spec = pl.BlockSpec((_LBLK, _D), lambda i: (i, 0))
    full_spec = pl.BlockSpec((_D, _D), lambda i: (0, 0))
    bias_spec = pl.BlockSpec((1, _D), lambda i: (0, 0))
    return pl.pallas_call(
        _layer1_body,
        grid=grid,
        in_specs=[row_spec, row_spec, row_spec, full_spec, full_spec, bias_spec],
        out_specs=row_spec,
        out_shape=jax.ShapeDtypeStruct((_N_PAD, _D), jnp.float32),
    )(feat_pad, p0, p1, w1a, w1b, b1r)


def _layer2_body(gf_ref, g0_ref, g1_ref, w2a_ref, w2b_ref, b2_ref, out_ref):
    h2 = jnp.dot(gf_ref[...], w2a_ref[...], preferred_element_type=jnp.float32)
    h2 = h2 + jnp.dot(g0_ref[...] + g1_ref[...], w2b_ref[...],
                      preferred_element_type=jnp.float32)
    h2 = h2 + b2_ref[...]
    nrm = jnp.sqrt(jnp.sum(h2 * h2, axis=1, keepdims=True))
    out_ref[...] = h2 / jnp.maximum(nrm, 1e-12)


def _layer2(gf, g0, g1, w2a, w2b, b2r):
    return pl.pallas_call(
        _layer2_body,
        out_shape=jax.ShapeDtypeStruct((_G, _D), jnp.float32),
    )(gf, g0, g1, w2a, w2b, b2r)


def kernel(feat_data, edge_index, inputs1, inputs2, neg, W1, b1, W2, b2):
    feat_pad = jnp.zeros((_N_PAD, _D), jnp.float32).at[:_N].set(feat_data)
    src = edge_index[0].astype(jnp.int32)
    dst = edge_index[1].astype(jnp.int32)
    pad_e = _E_PAD - _E
    src_pad = jnp.concatenate(
        [src, jnp.zeros((pad_e,), jnp.int32)]).reshape(_NW, _CPT, _CHUNK)
    dst_pad = jnp.concatenate(
        [dst, jnp.full((pad_e,), _N, jnp.int32)]).reshape(_NW, _CPT, _CHUNK)
    zeros = jnp.zeros((_N_PAD, _D), jnp.float32)

    p = _segsum(feat_pad, src_pad, dst_pad, zeros)

    w1t = W1.T
    h1 = _layer1(feat_pad, p[0], p[1], w1t[:_D], w1t[_D:],
                 b1.reshape(1, _D))

    q = _segsum(h1, src_pad, dst_pad, zeros)

    idx = jnp.concatenate([
        inputs1.astype(jnp.int32), inputs2.astype(jnp.int32),
        neg.astype(jnp.int32),
        jnp.zeros((_G - 2 * _B - _NEG,), jnp.int32)])
    gf, g0, g1 = _gather3(feat_pad, q[0], q[1], idx)

    w2t = W2.T
    out = _layer2(gf, g0, g1, w2t[:_D], w2t[_D:], b2.reshape(1, _D))
    return (out[:_B], out[_B:2 * _B], out[2 * _B:2 * _B + _NEG])


# speedup vs baseline: 2.2543x; 1.0088x over previous
import functools

import jax
import jax.numpy as jnp
from jax import lax
from jax.experimental import pallas as pl
from jax.experimental.pallas import tpu as pltpu
from jax.experimental.pallas import tpu_sc as plsc

_N = 10000
_E = 320000
_D = 128
_B = 1024
_NEG = 10

_NC = 2
_NS = 16
_NW = _NC * _NS
_CHUNK = 128
_N_PAD = 10240
_E_PAD = 327680
_CPT = _E_PAD // (_NW * _CHUNK)
_STRIPE = _N_PAD // _NS
_SUP = 8
_G = _NW * _CHUNK

_mesh = plsc.VectorSubcoreMesh(core_axis_name="c", subcore_axis_name="s")


@functools.partial(
    pl.kernel,
    out_type=jax.ShapeDtypeStruct((_NC, _N_PAD, _D), jnp.float32),
    mesh=_mesh,
    scratch_types=[
        pltpu.VMEM((_SUP, _CHUNK), jnp.int32),
        pltpu.VMEM((_SUP, _CHUNK), jnp.int32),
        pltpu.VMEM((_CHUNK, _D), jnp.float32),
        pltpu.VMEM((_CHUNK, _D), jnp.float32),
        pltpu.VMEM_SHARED((_N_PAD, _D), jnp.float32),
        pltpu.SemaphoreType.DMA,
        pltpu.SemaphoreType.DMA,
    ],
)
def _segsum(table_hbm, src_hbm, dst_hbm, zeros_hbm, out_hbm,
            src_v, dst_v, rows_a, rows_b, acc, sem_a, sem_b):
    c = lax.axis_index("c")
    s = lax.axis_index("s")
    w = c * _NS + s
    pltpu.sync_copy(zeros_hbm.at[pl.ds(s * _STRIPE, _STRIPE)],
                    acc.at[pl.ds(s * _STRIPE, _STRIPE)])
    plsc.subcore_barrier()

    def outer(t, carry):
        pltpu.sync_copy(src_hbm.at[w, pl.ds(t * _SUP, _SUP)], src_v)
        pltpu.sync_copy(dst_hbm.at[w, pl.ds(t * _SUP, _SUP)], dst_v)

        def body(i, carry2):
            j = 2 * i
            cp_a = pltpu.async_copy(table_hbm.at[src_v.at[j]], rows_a, sem_a)
            cp_b = pltpu.async_copy(table_hbm.at[src_v.at[j + 1]], rows_b,
                                    sem_b)
            cp_a.wait()
            pltpu.sync_copy(rows_a, acc.at[dst_v.at[j]], add=True)
            cp_b.wait()
            pltpu.sync_copy(rows_b, acc.at[dst_v.at[j + 1]], add=True)
            return carry2

        lax.fori_loop(0, _SUP // 2, body, 0)
        return carry

    lax.fori_loop(0, _CPT // _SUP, outer, 0)
    plsc.subcore_barrier()
    pltpu.sync_copy(acc.at[pl.ds(s * _STRIPE, _STRIPE)],
                    out_hbm.at[c, pl.ds(s * _STRIPE, _STRIPE)])


@functools.partial(
    pl.kernel,
    out_type=(
        jax.ShapeDtypeStruct((_G, _D), jnp.float32),
        jax.ShapeDtypeStruct((_G, _D), jnp.float32),
        jax.ShapeDtypeStruct((_G, _D), jnp.float32),
    ),
    mesh=_mesh,
    scratch_types=[
        pltpu.VMEM((_CHUNK,), jnp.int32),
        pltpu.VMEM((_CHUNK, _D), jnp.float32),
        pltpu.SemaphoreType.DMA,
    ],
)
def _gather3(feat_hbm, p0_hbm, p1_hbm, idx_hbm, gf_hbm, g0_hbm, g1_hbm,
             idx_v, rows_v, sem):
    c = lax.axis_index("c")
    s = lax.axis_index("s")
    w = c * _NS + s
    base = w * _CHUNK
    pltpu.sync_copy(idx_hbm.at[pl.ds(base, _CHUNK)], idx_v)
    pltpu.async_copy(feat_hbm.at[idx_v], rows_v, sem).wait()
    pltpu.sync_copy(rows_v, gf_hbm.at[pl.ds(base, _CHUNK)])
    pltpu.async_copy(p0_hbm.at[idx_v], rows_v, sem).wait()
    pltpu.sync_copy(rows_v, g0_hbm.at[pl.ds(base, _CHUNK)])
    pltpu.async_copy(p1_hbm.at[idx_v], rows_v, sem).wait()
    pltpu.sync_copy(rows_v, g1_hbm.at[pl.ds(base, _CHUNK)])


def _layer1_body(feat_ref, p0_ref, p1_ref, w1a_ref, w1b_ref, b1_ref, out_ref):
    ns = p0_ref[...] + p1_ref[...]
    acc = jnp.dot(feat_ref[...], w1a_ref[...],
                  preferred_element_type=jnp.float32)
    acc = acc + jnp.dot(ns, w1b_ref[...], preferred_element_type=jnp.float32)
    out_ref[...] = jnp.tanh(acc + b1_ref[...])


_LBLK = 1024


def _layer1(feat_pad, p0, p1, w1a, w1b, b1r):
    grid = (_N_PAD // _LBLK,)
    row_spec = pl.BlockSpec((_LBLK, _D), lambda i: (i, 0))
    full_spec = pl.BlockSpec((_D, _D), lambda i: (0, 0))
    bias_spec = pl.BlockSpec((1, _D), lambda i: (0, 0))
    return pl.pallas_call(
        _layer1_body,
        grid=grid,
        in_specs=[row_spec, row_spec, row_spec, full_spec, full_spec, bias_spec],
        out_specs=row_spec,
        out_shape=jax.ShapeDtypeStruct((_N_PAD, _D), jnp.float32),
    )(feat_pad, p0, p1, w1a, w1b, b1r)


def _layer2_body(gf_ref, g0_ref, g1_ref, w2a_ref, w2b_ref, b2_ref, out_ref):
    h2 = jnp.dot(gf_ref[...], w2a_ref[...], preferred_element_type=jnp.float32)
    h2 = h2 + jnp.dot(g0_ref[...] + g1_ref[...], w2b_ref[...],
                      preferred_element_type=jnp.float32)
    h2 = h2 + b2_ref[...]
    nrm = jnp.sqrt(jnp.sum(h2 * h2, axis=1, keepdims=True))
    out_ref[...] = h2 / jnp.maximum(nrm, 1e-12)


def _layer2(gf, g0, g1, w2a, w2b, b2r):
    return pl.pallas_call(
        _layer2_body,
        out_shape=jax.ShapeDtypeStruct((_G, _D), jnp.float32),
    )(gf, g0, g1, w2a, w2b, b2r)


def kernel(feat_data, edge_index, inputs1, inputs2, neg, W1, b1, W2, b2):
    feat_pad = jnp.zeros((_N_PAD, _D), jnp.float32).at[:_N].set(feat_data)
    src = edge_index[0].astype(jnp.int32)
    dst = edge_index[1].astype(jnp.int32)
    pad_e = _E_PAD - _E
    src_pad = jnp.concatenate(
        [src, jnp.zeros((pad_e,), jnp.int32)]).reshape(_NW, _CPT, _CHUNK)
    dst_pad = jnp.concatenate(
        [dst, jnp.full((pad_e,), _N, jnp.int32)]).reshape(_NW, _CPT, _CHUNK)
    zeros = jnp.zeros((_N_PAD, _D), jnp.float32)

    p = _segsum(feat_pad, src_pad, dst_pad, zeros)

    w1t = W1.T
    h1 = _layer1(feat_pad, p[0], p[1], w1t[:_D], w1t[_D:],
                 b1.reshape(1, _D))

    q = _segsum(h1, src_pad, dst_pad, zeros)

    idx = jnp.concatenate([
        inputs1.astype(jnp.int32), inputs2.astype(jnp.int32),
        neg.astype(jnp.int32),
        jnp.zeros((_G - 2 * _B - _NEG,), jnp.int32)])
    gf, g0, g1 = _gather3(feat_pad, q[0], q[1], idx)

    w2t = W2.T
    out = _layer2(gf, g0, g1, w2t[:_D], w2t[_D:], b2.reshape(1, _D))
    return (out[:_B], out[_B:2 * _B], out[2 * _B:2 * _B + _NEG])


# speedup vs baseline: 3.0862x; 1.3690x over previous
import functools

import jax
import jax.numpy as jnp
from jax import lax
from jax.experimental import pallas as pl
from jax.experimental.pallas import tpu as pltpu
from jax.experimental.pallas import tpu_sc as plsc

_N = 10000
_E = 320000
_D = 128
_B = 1024
_NEG = 10

_NC = 2
_NS = 16
_CHUNK = 128
_N_PAD = 10240
_E_PAD = 327680
_NCHUNK = _E_PAD // _CHUNK
_N0 = 120
_N1 = 40
_STRIPE = _N_PAD // _NS
_SUP = 8
_G = 4096

_mesh = plsc.VectorSubcoreMesh(core_axis_name="c", subcore_axis_name="s")


@functools.partial(
    pl.kernel,
    out_type=jax.ShapeDtypeStruct((_NC, _N_PAD, _D), jnp.float32),
    mesh=_mesh,
    scratch_types=[
        pltpu.VMEM((_SUP, _CHUNK), jnp.int32),
        pltpu.VMEM((_SUP, _CHUNK), jnp.int32),
        pltpu.VMEM((_CHUNK, _D), jnp.float32),
        pltpu.VMEM((_CHUNK, _D), jnp.float32),
        pltpu.VMEM_SHARED((_N_PAD, _D), jnp.float32),
        pltpu.SemaphoreType.DMA,
        pltpu.SemaphoreType.DMA,
    ],
)
def _segsum(table_hbm, src_hbm, dst_hbm, out_hbm,
            src_v, dst_v, rows_a, rows_b, acc, sem_a, sem_b):
    c = lax.axis_index("c")
    s = lax.axis_index("s")
    def zrow(i, carry):
        for k in range(_D // 16):
            rows_a[i, pl.ds(16 * k, 16)] = jnp.zeros((16,), jnp.float32)
        return carry

    lax.fori_loop(0, _CHUNK, zrow, 0)
    for m in range(_STRIPE // _CHUNK):
        pltpu.sync_copy(rows_a, acc.at[pl.ds(s * _STRIPE + m * _CHUNK, _CHUNK)])
    plsc.subcore_barrier()

    base_chunk = jnp.where(c == 0, s * _N0, _NS * _N0 + s * _N1)
    nblocks = jnp.where(c == 0, _N0 // _SUP, _N1 // _SUP)

    def outer(t, carry):
        blk = base_chunk + t * _SUP
        pltpu.sync_copy(src_hbm.at[pl.ds(blk, _SUP)], src_v)
        pltpu.sync_copy(dst_hbm.at[pl.ds(blk, _SUP)], dst_v)

        def body(i, carry2):
            j = 2 * i
            cp_a = pltpu.async_copy(table_hbm.at[src_v.at[j]], rows_a, sem_a)
            cp_b = pltpu.async_copy(table_hbm.at[src_v.at[j + 1]], rows_b,
                                    sem_b)
            cp_a.wait()
            pltpu.sync_copy(rows_a, acc.at[dst_v.at[j]], add=True)
            cp_b.wait()
            pltpu.sync_copy(rows_b, acc.at[dst_v.at[j + 1]], add=True)
            return carry2

        lax.fori_loop(0, _SUP // 2, body, 0)
        return carry

    lax.fori_loop(0, nblocks, outer, 0)
    plsc.subcore_barrier()
    pltpu.sync_copy(acc.at[pl.ds(s * _STRIPE, _STRIPE)],
                    out_hbm.at[c, pl.ds(s * _STRIPE, _STRIPE)])


@functools.partial(
    pl.kernel,
    out_type=(
        jax.ShapeDtypeStruct((_G, _D), jnp.float32),
        jax.ShapeDtypeStruct((_G, _D), jnp.float32),
        jax.ShapeDtypeStruct((_G, _D), jnp.float32),
    ),
    mesh=_mesh,
    scratch_types=[
        pltpu.VMEM((2, _CHUNK), jnp.int32),
        pltpu.VMEM((_CHUNK, _D), jnp.float32),
        pltpu.SemaphoreType.DMA,
    ],
)
def _gather3(feat_hbm, p0_hbm, p1_hbm, idx_hbm, gf_hbm, g0_hbm, g1_hbm,
             idx_v, rows_v, sem):
    c = lax.axis_index("c")
    s = lax.axis_index("s")

    @pl.when(c == 0)
    def _():
        pltpu.sync_copy(idx_hbm.at[s], idx_v)
        for q in range(2):
            base = (2 * s + q) * _CHUNK
            for tbl, dst in ((feat_hbm, gf_hbm), (p0_hbm, g0_hbm),
                             (p1_hbm, g1_hbm)):
                pltpu.async_copy(tbl.at[idx_v.at[q]], rows_v, sem).wait()
                pltpu.sync_copy(rows_v, dst.at[pl.ds(base, _CHUNK)])


def _layer1_body(feat_ref, p0_ref, p1_ref, w1a_ref, w1b_ref, b1_ref, out_ref):
    ns = p0_ref[...] + p1_ref[...]
    acc = jnp.dot(feat_ref[...], w1a_ref[...],
                  preferred_element_type=jnp.float32)
    acc = acc + jnp.dot(ns, w1b_ref[...], preferred_element_type=jnp.float32)
    out_ref[...] = jnp.tanh(acc + b1_ref[...])


_LBLK = 1024


def _layer1(feat_pad, p0, p1, w1a, w1b, b1r):
    grid = (_N_PAD // _LBLK,)
    row_spec = pl.BlockSpec((_LBLK, _D), lambda i: (i, 0))
    full_spec = pl.BlockSpec((_D, _D), lambda i: (0, 0))
    bias_spec = pl.BlockSpec((1, _D), lambda i: (0, 0))
    return pl.pallas_call(
        _layer1_body,
        grid=grid,
        in_specs=[row_spec, row_spec, row_spec, full_spec, full_spec, bias_spec],
        out_specs=row_spec,
        out_shape=jax.ShapeDtypeStruct((_N_PAD, _D), jnp.float32),
    )(feat_pad, p0, p1, w1a, w1b, b1r)


def _layer2_body(gf_ref, g0_ref, g1_ref, w2a_ref, w2b_ref, b2_ref, out_ref):
    h2 = jnp.dot(gf_ref[...], w2a_ref[...], preferred_element_type=jnp.float32)
    h2 = h2 + jnp.dot(g0_ref[...] + g1_ref[...], w2b_ref[...],
                      preferred_element_type=jnp.float32)
    h2 = h2 + b2_ref[...]
    nrm = jnp.sqrt(jnp.sum(h2 * h2, axis=1, keepdims=True))
    out_ref[...] = h2 / jnp.maximum(nrm, 1e-12)


def _layer2(gf, g0, g1, w2a, w2b, b2r):
    return pl.pallas_call(
        _layer2_body,
        out_shape=jax.ShapeDtypeStruct((_G, _D), jnp.float32),
    )(gf, g0, g1, w2a, w2b, b2r)


def kernel(feat_data, edge_index, inputs1, inputs2, neg, W1, b1, W2, b2):
    feat_pad = jnp.zeros((_N_PAD, _D), jnp.float32).at[:_N].set(feat_data)
    src = edge_index[0].astype(jnp.int32)
    dst = edge_index[1].astype(jnp.int32)
    pad_e = _E_PAD - _E
    src_pad = jnp.concatenate(
        [src, jnp.zeros((pad_e,), jnp.int32)]).reshape(_NCHUNK, _CHUNK)
    dst_pad = jnp.concatenate(
        [dst, jnp.full((pad_e,), _N, jnp.int32)]).reshape(_NCHUNK, _CHUNK)

    p = _segsum(feat_pad, src_pad, dst_pad)

    w1t = W1.T
    h1 = _layer1(feat_pad, p[0], p[1], w1t[:_D], w1t[_D:],
                 b1.reshape(1, _D))

    q = _segsum(h1, src_pad, dst_pad)

    idx = jnp.concatenate([
        inputs1.astype(jnp.int32), inputs2.astype(jnp.int32),
        neg.astype(jnp.int32),
        jnp.zeros((_G - 2 * _B - _NEG,), jnp.int32)]).reshape(_NS, 2, _CHUNK)
    gf, g0, g1 = _gather3(feat_pad, q[0], q[1], idx)

    w2t = W2.T
    out = _layer2(gf, g0, g1, w2t[:_D], w2t[_D:], b2.reshape(1, _D))
    return (out[:_B], out[_B:2 * _B], out[2 * _B:2 * _B + _NEG])


# speedup vs baseline: 3.2264x; 1.0454x over previous
import functools

import jax
import jax.numpy as jnp
from jax import lax
from jax.experimental import pallas as pl
from jax.experimental.pallas import tpu as pltpu
from jax.experimental.pallas import tpu_sc as plsc

_N = 10000
_E = 320000
_D = 128
_B = 1024
_NEG = 10

_NC = 2
_NS = 16
_CHUNK = 128
_N_PAD = 10240
_E_PAD = 327680
_NCHUNK = _E_PAD // _CHUNK
_N0 = 144
_N1 = 16
_STRIPE = _N_PAD // _NS
_SUP = 8
_G = 4096

_mesh = plsc.VectorSubcoreMesh(core_axis_name="c", subcore_axis_name="s")


@functools.partial(
    pl.kernel,
    out_type=jax.ShapeDtypeStruct((_NC, _N_PAD, _D), jnp.float32),
    mesh=_mesh,
    scratch_types=[
        pltpu.VMEM((_SUP, _CHUNK), jnp.int32),
        pltpu.VMEM((_SUP, _CHUNK), jnp.int32),
        pltpu.VMEM((_CHUNK, _D), jnp.float32),
        pltpu.VMEM((_CHUNK, _D), jnp.float32),
        pltpu.VMEM_SHARED((_N_PAD, _D), jnp.float32),
        pltpu.SemaphoreType.DMA,
        pltpu.SemaphoreType.DMA,
    ],
)
def _segsum(table_hbm, src_hbm, dst_hbm, out_hbm,
            src_v, dst_v, rows_a, rows_b, acc, sem_a, sem_b):
    c = lax.axis_index("c")
    s = lax.axis_index("s")
    def zrow(i, carry):
        for k in range(_D // 16):
            rows_a[i, pl.ds(16 * k, 16)] = jnp.zeros((16,), jnp.float32)
        return carry

    lax.fori_loop(0, _CHUNK, zrow, 0)
    for m in range(_STRIPE // _CHUNK):
        pltpu.sync_copy(rows_a, acc.at[pl.ds(s * _STRIPE + m * _CHUNK, _CHUNK)])
    plsc.subcore_barrier()

    base_chunk = jnp.where(c == 0, s * _N0, _NS * _N0 + s * _N1)
    nblocks = jnp.where(c == 0, _N0 // _SUP, _N1 // _SUP)

    def outer(t, carry):
        blk = base_chunk + t * _SUP
        pltpu.sync_copy(src_hbm.at[pl.ds(blk, _SUP)], src_v)
        pltpu.sync_copy(dst_hbm.at[pl.ds(blk, _SUP)], dst_v)

        def body(i, carry2):
            j = 2 * i
            cp_a = pltpu.async_copy(table_hbm.at[src_v.at[j]], rows_a, sem_a)
            cp_b = pltpu.async_copy(table_hbm.at[src_v.at[j + 1]], rows_b,
                                    sem_b)
            cp_a.wait()
            pltpu.sync_copy(rows_a, acc.at[dst_v.at[j]], add=True)
            cp_b.wait()
            pltpu.sync_copy(rows_b, acc.at[dst_v.at[j + 1]], add=True)
            return carry2

        lax.fori_loop(0, _SUP // 2, body, 0)
        return carry

    lax.fori_loop(0, nblocks, outer, 0)
    plsc.subcore_barrier()
    pltpu.sync_copy(acc.at[pl.ds(s * _STRIPE, _STRIPE)],
                    out_hbm.at[c, pl.ds(s * _STRIPE, _STRIPE)])


@functools.partial(
    pl.kernel,
    out_type=(
        jax.ShapeDtypeStruct((_G, _D), jnp.float32),
        jax.ShapeDtypeStruct((_G, _D), jnp.float32),
        jax.ShapeDtypeStruct((_G, _D), jnp.float32),
    ),
    mesh=_mesh,
    scratch_types=[
        pltpu.VMEM((2, _CHUNK), jnp.int32),
        pltpu.VMEM((_CHUNK, _D), jnp.float32),
        pltpu.SemaphoreType.DMA,
    ],
)
def _gather3(feat_hbm, p0_hbm, p1_hbm, idx_hbm, gf_hbm, g0_hbm, g1_hbm,
             idx_v, rows_v, sem):
    c = lax.axis_index("c")
    s = lax.axis_index("s")

    @pl.when(c == 0)
    def _():
        pltpu.sync_copy(idx_hbm.at[s], idx_v)
        for q in range(2):
            base = (2 * s + q) * _CHUNK
            for tbl, dst in ((feat_hbm, gf_hbm), (p0_hbm, g0_hbm),
                             (p1_hbm, g1_hbm)):
                pltpu.async_copy(tbl.at[idx_v.at[q]], rows_v, sem).wait()
                pltpu.sync_copy(rows_v, dst.at[pl.ds(base, _CHUNK)])


def _layer1_body(feat_ref, p0_ref, p1_ref, w1a_ref, w1b_ref, b1_ref, out_ref):
    ns = p0_ref[...] + p1_ref[...]
    acc = jnp.dot(feat_ref[...], w1a_ref[...],
                  preferred_element_type=jnp.float32)
    acc = acc + jnp.dot(ns, w1b_ref[...], preferred_element_type=jnp.float32)
    out_ref[...] = jnp.tanh(acc + b1_ref[...])


_LBLK = 1024


def _layer1(feat_pad, p0, p1, w1a, w1b, b1r):
    grid = (_N_PAD // _LBLK,)
    row_spec = pl.BlockSpec((_LBLK, _D), lambda i: (i, 0))
    full_spec = pl.BlockSpec((_D, _D), lambda i: (0, 0))
    bias_spec = pl.BlockSpec((1, _D), lambda i: (0, 0))
    return pl.pallas_call(
        _layer1_body,
        grid=grid,
        in_specs=[row_spec, row_spec, row_spec, full_spec, full_spec, bias_spec],
        out_specs=row_spec,
        out_shape=jax.ShapeDtypeStruct((_N_PAD, _D), jnp.float32),
    )(feat_pad, p0, p1, w1a, w1b, b1r)


def _layer2_body(gf_ref, g0_ref, g1_ref, w2a_ref, w2b_ref, b2_ref, out_ref):
    h2 = jnp.dot(gf_ref[...], w2a_ref[...], preferred_element_type=jnp.float32)
    h2 = h2 + jnp.dot(g0_ref[...] + g1_ref[...], w2b_ref[...],
                      preferred_element_type=jnp.float32)
    h2 = h2 + b2_ref[...]
    nrm = jnp.sqrt(jnp.sum(h2 * h2, axis=1, keepdims=True))
    out_ref[...] = h2 / jnp.maximum(nrm, 1e-12)


def _layer2(gf, g0, g1, w2a, w2b, b2r):
    return pl.pallas_call(
        _layer2_body,
        out_shape=jax.ShapeDtypeStruct((_G, _D), jnp.float32),
    )(gf, g0, g1, w2a, w2b, b2r)


def kernel(feat_data, edge_index, inputs1, inputs2, neg, W1, b1, W2, b2):
    feat_pad = jnp.zeros((_N_PAD, _D), jnp.float32).at[:_N].set(feat_data)
    src = edge_index[0].astype(jnp.int32)
    dst = edge_index[1].astype(jnp.int32)
    pad_e = _E_PAD - _E
    src_pad = jnp.concatenate(
        [src, jnp.zeros((pad_e,), jnp.int32)]).reshape(_NCHUNK, _CHUNK)
    dst_pad = jnp.concatenate(
        [dst, jnp.full((pad_e,), _N, jnp.int32)]).reshape(_NCHUNK, _CHUNK)

    p = _segsum(feat_pad, src_pad, dst_pad)

    w1t = W1.T
    h1 = _layer1(feat_pad, p[0], p[1], w1t[:_D], w1t[_D:],
                 b1.reshape(1, _D))

    q = _segsum(h1, src_pad, dst_pad)

    idx = jnp.concatenate([
        inputs1.astype(jnp.int32), inputs2.astype(jnp.int32),
        neg.astype(jnp.int32),
        jnp.zeros((_G - 2 * _B - _NEG,), jnp.int32)]).reshape(_NS, 2, _CHUNK)
    gf, g0, g1 = _gather3(feat_pad, q[0], q[1], idx)

    w2t = W2.T
    out = _layer2(gf, g0, g1, w2t[:_D], w2t[_D:], b2.reshape(1, _D))
    return (out[:_B], out[_B:2 * _B], out[2 * _B:2 * _B + _NEG])
